# Initial kernel scaffold; baseline (speedup 1.0000x reference)
import jax
import jax.numpy as jnp
from jax.experimental import pallas as pl


def kernel(inputs, embeddings, w):
    raise NotImplementedError("write your pallas kernel here")


# speedup vs baseline: 3.0590x; 3.0590x over previous
import functools

import jax
import jax.numpy as jnp
from jax import lax
from jax.experimental import pallas as pl
from jax.experimental.pallas import tpu as pltpu
from jax.experimental.pallas import tpu_sc as plsc


def _mul_body(e_ref, w_ref, o_ref):
    o_ref[...] = e_ref[...] * w_ref[...]


def _dense_mul(embeddings, w):
    V, D = embeddings.shape
    rows = 4000
    grid = V // rows
    spec = pl.BlockSpec((rows, D), lambda i: (i, 0))
    return pl.pallas_call(
        _mul_body,
        out_shape=jax.ShapeDtypeStruct((V, D), embeddings.dtype),
        grid=(grid,),
        in_specs=[spec, spec],
        out_specs=spec,
    )(embeddings, w)


_CH = 128


def _make_gather(V, D, B):
    info = plsc.get_sparse_core_info()
    NC, NS = info.num_cores, info.num_subcores
    NW = NC * NS
    bpw = B // NW
    CH = _CH
    nch = bpw // CH
    assert B % NW == 0 and bpw % CH == 0

    mesh = plsc.VectorSubcoreMesh(core_axis_name="c", subcore_axis_name="s")

    @functools.partial(
        pl.kernel, mesh=mesh,
        out_type=jax.ShapeDtypeStruct((B, D), jnp.float32),
        compiler_params=pltpu.CompilerParams(use_tc_tiling_on_sc=False),
        scratch_types=[
            pltpu.VMEM((nch, CH), jnp.int32),
            pltpu.VMEM((CH, D), jnp.float32),
            pltpu.SemaphoreType.DMA,
        ],
    )
    def gather(table_hbm, idx_hbm, out_hbm, idx_v, rows_v, sem):
        wid = lax.axis_index("s") * NC + lax.axis_index("c")
        base = wid * bpw
        pltpu.sync_copy(idx_hbm.at[wid], idx_v)

        def body(i, _):
            pltpu.async_copy(table_hbm.at[idx_v.at[i]], rows_v, sem).wait()
            pltpu.sync_copy(rows_v, out_hbm.at[pl.ds(base + i * CH, CH)])
            return 0

        lax.fori_loop(0, nch, body, 0)

    return gather


def kernel(inputs, embeddings, w):
    Bt, S = inputs.shape
    V, D = embeddings.shape
    B = Bt * S
    emb = _dense_mul(embeddings, w)
    info = plsc.get_sparse_core_info()
    NW = info.num_cores * info.num_subcores
    ids3d = inputs.reshape(NW, B // (NW * _CH), _CH).astype(jnp.int32)
    out = _make_gather(V, D, B)(emb, ids3d)
    return out.reshape(Bt, S, D)


# speedup vs baseline: 3.5006x; 1.1444x over previous
import functools

import jax
import jax.numpy as jnp
from jax import lax
from jax.experimental import pallas as pl
from jax.experimental.pallas import tpu as pltpu
from jax.experimental.pallas import tpu_sc as plsc


def _mul_body(e_ref, w_ref, o_ref):
    o_ref[...] = e_ref[...] * w_ref[...]


def _dense_mul(e2, w2):
    R, C = e2.shape
    rows = 2000
    spec = pl.BlockSpec((rows, C), lambda i: (i, 0))
    return pl.pallas_call(
        _mul_body,
        out_shape=jax.ShapeDtypeStruct((R, C), e2.dtype),
        grid=(R // rows,),
        in_specs=[spec, spec],
        out_specs=spec,
    )(e2, w2)


_CH = 128


def _make_gather(V, D, B):
    info = plsc.get_sparse_core_info()
    NC, NS = info.num_cores, info.num_subcores
    NW = NC * NS
    bpw = B // NW
    CH = _CH
    nch = bpw // CH
    assert B % NW == 0 and bpw % CH == 0
    CHB = CH * D * 4

    mesh = plsc.VectorSubcoreMesh(core_axis_name="c", subcore_axis_name="s")

    @functools.partial(
        pl.kernel, mesh=mesh,
        out_type=jax.ShapeDtypeStruct((B, D), jnp.float32),
        compiler_params=pltpu.CompilerParams(use_tc_tiling_on_sc=False),
        scratch_types=[
            pltpu.VMEM((nch, CH), jnp.int32),
            pltpu.VMEM((CH, D), jnp.float32),
            pltpu.VMEM((CH, D), jnp.float32),
            pltpu.SemaphoreType.DMA,
            pltpu.SemaphoreType.DMA,
            pltpu.SemaphoreType.DMA,
            pltpu.SemaphoreType.DMA,
        ],
    )
    def gather(table_hbm, idx_hbm, out_hbm, idx_v, buf_a, buf_b,
               gsem_a, gsem_b, wsem_a, wsem_b):
        wid = lax.axis_index("s") * NC + lax.axis_index("c")
        base = wid * bpw
        pltpu.sync_copy(idx_hbm.at[wid], idx_v)

        pltpu.async_copy(table_hbm.at[idx_v.at[0]], buf_a, gsem_a)

        def wait_write(buf_, wsem_):
            pltpu.make_async_copy(
                buf_, out_hbm.at[pl.ds(base, CH)], wsem_).wait()

        def wait_gather(buf_, gsem_):
            pltpu.make_async_copy(
                table_hbm.at[pl.ds(0, CH)], buf_, gsem_).wait()

        def step(k, buf, gsem, wsem, obuf, ogsem, owsem):
            @pl.when(k + 1 < nch)
            def _():
                @pl.when(k >= 1)
                def _():
                    wait_write(obuf, owsem)

                pltpu.async_copy(table_hbm.at[idx_v.at[k + 1]], obuf, ogsem)

            wait_gather(buf, gsem)
            pltpu.async_copy(buf, out_hbm.at[pl.ds(base + k * CH, CH)], wsem)

        def body(k, _):
            @pl.when(k % 2 == 0)
            def _():
                step(k, buf_a, gsem_a, wsem_a, buf_b, gsem_b, wsem_b)

            @pl.when(k % 2 == 1)
            def _():
                step(k, buf_b, gsem_b, wsem_b, buf_a, gsem_a, wsem_a)

            return 0

        lax.fori_loop(0, nch, body, 0)
        wait_write(buf_a, wsem_a)
        wait_write(buf_b, wsem_b)

    return gather


def kernel(inputs, embeddings, w):
    Bt, S = inputs.shape
    V, D = embeddings.shape
    B = Bt * S
    e2 = embeddings.reshape(V // 2, 2 * D)
    w2 = w.reshape(V // 2, 2 * D)
    emb = _dense_mul(e2, w2).reshape(V, D)
    info = plsc.get_sparse_core_info()
    NW = info.num_cores * info.num_subcores
    ids3d = inputs.reshape(NW, B // (NW * _CH), _CH).astype(jnp.int32)
    out = _make_gather(V, D, B)(emb, ids3d)
    return out.reshape(Bt, S, D)


# speedup vs baseline: 4.3686x; 1.2479x over previous
import functools

import jax
import jax.numpy as jnp
from jax import lax
from jax.experimental import pallas as pl
from jax.experimental.pallas import tpu as pltpu
from jax.experimental.pallas import tpu_sc as plsc


def _mul_t_body(e_ref, w_ref, o_ref):
    o_ref[...] = (e_ref[...] * w_ref[...]).T


def _dense_mul_t(eT, wT):
    D_, V_ = eT.shape
    cols = 12800
    grid = (V_ + cols - 1) // cols
    in_spec = pl.BlockSpec((D_, cols), lambda i: (0, i))
    out_spec = pl.BlockSpec((cols, D_), lambda i: (i, 0))
    return pl.pallas_call(
        _mul_t_body,
        out_shape=jax.ShapeDtypeStruct((V_, D_), eT.dtype),
        grid=(grid,),
        in_specs=[in_spec, in_spec],
        out_specs=out_spec,
    )(eT, wT)


_CH = 128


def _make_gather(V, D, B):
    info = plsc.get_sparse_core_info()
    NC, NS = info.num_cores, info.num_subcores
    NW = NC * NS
    bpw = B // NW
    CH = _CH
    nch = bpw // CH
    assert B % NW == 0 and bpw % CH == 0
    CHB = CH * D * 4

    mesh = plsc.VectorSubcoreMesh(core_axis_name="c", subcore_axis_name="s")

    @functools.partial(
        pl.kernel, mesh=mesh,
        out_type=jax.ShapeDtypeStruct((B, D), jnp.float32),
        compiler_params=pltpu.CompilerParams(use_tc_tiling_on_sc=False),
        scratch_types=[
            pltpu.VMEM((nch, CH), jnp.int32),
            pltpu.VMEM((CH, D), jnp.float32),
            pltpu.VMEM((CH, D), jnp.float32),
            pltpu.SemaphoreType.DMA,
            pltpu.SemaphoreType.DMA,
            pltpu.SemaphoreType.DMA,
            pltpu.SemaphoreType.DMA,
        ],
    )
    def gather(table_hbm, idx_hbm, out_hbm, idx_v, buf_a, buf_b,
               gsem_a, gsem_b, wsem_a, wsem_b):
        wid = lax.axis_index("s") * NC + lax.axis_index("c")
        base = wid * bpw
        pltpu.sync_copy(idx_hbm.at[wid], idx_v)

        pltpu.async_copy(table_hbm.at[idx_v.at[0]], buf_a, gsem_a)

        def wait_write(buf_, wsem_):
            pltpu.make_async_copy(
                buf_, out_hbm.at[pl.ds(base, CH)], wsem_).wait()

        def wait_gather(buf_, gsem_):
            pltpu.make_async_copy(
                table_hbm.at[pl.ds(0, CH)], buf_, gsem_).wait()

        def step(k, buf, gsem, wsem, obuf, ogsem, owsem):
            @pl.when(k + 1 < nch)
            def _():
                @pl.when(k >= 1)
                def _():
                    wait_write(obuf, owsem)

                pltpu.async_copy(table_hbm.at[idx_v.at[k + 1]], obuf, ogsem)

            wait_gather(buf, gsem)
            pltpu.async_copy(buf, out_hbm.at[pl.ds(base + k * CH, CH)], wsem)

        def body(k, _):
            @pl.when(k % 2 == 0)
            def _():
                step(k, buf_a, gsem_a, wsem_a, buf_b, gsem_b, wsem_b)

            @pl.when(k % 2 == 1)
            def _():
                step(k, buf_b, gsem_b, wsem_b, buf_a, gsem_a, wsem_a)

            return 0

        lax.fori_loop(0, nch, body, 0)
        wait_write(buf_a, wsem_a)
        wait_write(buf_b, wsem_b)

    return gather


def kernel(inputs, embeddings, w):
    Bt, S = inputs.shape
    V, D = embeddings.shape
    B = Bt * S
    emb = _dense_mul_t(embeddings.T, w.T)
    info = plsc.get_sparse_core_info()
    NW = info.num_cores * info.num_subcores
    ids3d = inputs.reshape(NW, B // (NW * _CH), _CH).astype(jnp.int32)
    out = _make_gather(V, D, B)(emb, ids3d)
    return out.reshape(Bt, S, D)
